# baseline (speedup 1.0000x reference)
import functools

import jax
import jax.numpy as jnp
from jax import lax
from jax.experimental import pallas as pl
from jax.experimental.pallas import tpu as pltpu
from jax.experimental.pallas import tpu_sc as plsc

N_REL = 100
ARITY = 32
DIM = 128
BATCH = 16384
PAIRS = ARITY * (ARITY - 1)

_IDX_CHUNK = 128


@functools.cache
def _build():
    info = plsc.get_sparse_core_info()
    nc, ns = info.num_cores, info.num_subcores
    nw = nc * ns
    b_per_w = BATCH // nw
    n_chunk = b_per_w // _IDX_CHUNK
    n_grp = b_per_w // 16
    mesh = plsc.VectorSubcoreMesh(core_axis_name="c", subcore_axis_name="s")

    @functools.partial(
        pl.kernel,
        mesh=mesh,
        out_type=jax.ShapeDtypeStruct((BATCH, DIM), jnp.float32),
        scratch_types=[
            pltpu.VMEM((3, b_per_w), jnp.int32),
            pltpu.VMEM((n_chunk, _IDX_CHUNK), jnp.int32),
            pltpu.VMEM((b_per_w, DIM), jnp.float32),
            pltpu.SemaphoreType.DMA((2, n_chunk)),
        ],
    )
    def sc_gather(br_hbm, table_hbm, out_hbm, br_v, idx_v, rows_v, sems):
        wid = lax.axis_index("s") * nc + lax.axis_index("c")
        base = wid * b_per_w
        grp_per_chunk = _IDX_CHUNK // 16
        pltpu.sync_copy(br_hbm.at[:, pl.ds(base, b_per_w)], br_v)

        def idx_grp(j, gg):
            g = j * grp_per_chunk + gg
            i_r = br_v[0, pl.ds(g * 16, 16)]
            i1 = br_v[1, pl.ds(g * 16, 16)]
            i2 = br_v[2, pl.ds(g * 16, 16)]
            i2_adj = jnp.where(i2 > i1, i2 - 1, i2)
            flat = i_r * PAIRS + i1 * (ARITY - 1) + i2_adj
            idx_v[j, pl.ds(gg * 16, 16)] = flat

        def fire_gather(j):
            return pltpu.async_copy(
                table_hbm.at[idx_v.at[j]],
                rows_v.at[pl.ds(j * _IDX_CHUNK, _IDX_CHUNK)],
                sems.at[0, j],
            )

        gathers = []
        for j in range(n_chunk):
            for gg in range(grp_per_chunk):
                idx_grp(j, gg)
            gathers.append(fire_gather(j))
        for c in gathers:
            c.wait()
        pltpu.sync_copy(rows_v, out_hbm.at[pl.ds(base, b_per_w)])

    return sc_gather


def kernel(batch_rels, table):
    return _build()(batch_rels.T, table)


# speedup vs baseline: 1.0189x; 1.0189x over previous
import functools

import jax
import jax.numpy as jnp
from jax import lax
from jax.experimental import pallas as pl
from jax.experimental.pallas import tpu as pltpu
from jax.experimental.pallas import tpu_sc as plsc

N_REL = 100
ARITY = 32
DIM = 128
BATCH = 16384
PAIRS = ARITY * (ARITY - 1)

_IDX_CHUNK = 512


@functools.cache
def _build():
    info = plsc.get_sparse_core_info()
    nc, ns = info.num_cores, info.num_subcores
    nw = nc * ns
    b_per_w = BATCH // nw
    n_chunk = b_per_w // _IDX_CHUNK
    n_grp = b_per_w // 16
    mesh = plsc.VectorSubcoreMesh(core_axis_name="c", subcore_axis_name="s")

    @functools.partial(
        pl.kernel,
        mesh=mesh,
        out_type=jax.ShapeDtypeStruct((BATCH, DIM), jnp.float32),
        scratch_types=[
            pltpu.VMEM((3, b_per_w), jnp.int32),
            pltpu.VMEM((n_chunk, _IDX_CHUNK), jnp.int32),
            pltpu.VMEM((b_per_w, DIM), jnp.float32),
            pltpu.SemaphoreType.DMA((2, n_chunk)),
        ],
    )
    def sc_gather(br_hbm, table_hbm, out_hbm, br_v, idx_v, rows_v, sems):
        wid = lax.axis_index("s") * nc + lax.axis_index("c")
        base = wid * b_per_w
        grp_per_chunk = _IDX_CHUNK // 16
        pltpu.sync_copy(br_hbm.at[:, pl.ds(base, b_per_w)], br_v)

        def idx_grp(j, gg):
            g = j * grp_per_chunk + gg
            i_r = br_v[0, pl.ds(g * 16, 16)]
            i1 = br_v[1, pl.ds(g * 16, 16)]
            i2 = br_v[2, pl.ds(g * 16, 16)]
            i2_adj = jnp.where(i2 > i1, i2 - 1, i2)
            flat = i_r * PAIRS + i1 * (ARITY - 1) + i2_adj
            idx_v[j, pl.ds(gg * 16, 16)] = flat

        def fire_gather(j):
            return pltpu.async_copy(
                table_hbm.at[idx_v.at[j]],
                rows_v.at[pl.ds(j * _IDX_CHUNK, _IDX_CHUNK)],
                sems.at[0, j],
            )

        gathers = []
        for j in range(n_chunk):
            for gg in range(grp_per_chunk):
                idx_grp(j, gg)
            gathers.append(fire_gather(j))
        for c in gathers:
            c.wait()
        pltpu.sync_copy(rows_v, out_hbm.at[pl.ds(base, b_per_w)])

    return sc_gather


def kernel(batch_rels, table):
    return _build()(batch_rels.T, table)
